# baseline (speedup 1.0000x reference)
import functools

import jax
import jax.numpy as jnp
from jax import lax
from jax.experimental import pallas as pl
from jax.experimental.pallas import tpu as pltpu


def _slot_rnn_fwd(tok_sm, table_hbm, wih0_ref, bias0i_ref, w0_ref, w1_ref,
                  ghb_ref, wih1_ref, bias1i_ref, wlin_ref, blin_ref,
                  o_ref, emb_ref, gi_ref, h_ref, sem):
    i = pl.program_id(0)
    Bs, T, C = o_ref.shape
    E = emb_ref.shape[1] * emb_ref.shape[2]
    S = emb_ref.shape[1]
    H3 = wih0_ref.shape[1]
    H = H3 // 3
    M = Bs * T

    for t in range(T):
        for b in range(Bs):
            tok = tok_sm[i * Bs + b, t]
            pltpu.make_async_copy(
                table_hbm.at[tok], emb_ref.at[t * Bs + b], sem).start()
    pltpu.make_async_copy(
        emb_ref.at[pl.ds(0, M)], emb_ref.at[pl.ds(0, M)], sem).wait()

    gi = bias0i_ref[...]
    for s in range(S):
        gi = gi + jnp.dot(emb_ref[:, s, :],
                          wih0_ref[pl.ds(128 * s, 128), :],
                          preferred_element_type=jnp.float32)
    gi_ref[...] = gi

    def gru_steps(w_ref, gb):
        w = w_ref[...]
        h = None
        for t in range(T):
            gi = gi_ref[pl.ds(t * Bs, Bs), :]
            if h is None:
                gh = jnp.broadcast_to(gb, (Bs, H3))
            else:
                gh = jnp.dot(h, w, preferred_element_type=jnp.float32) + gb
            rz = jax.nn.sigmoid(gi[:, :2 * H] + gh[:, :2 * H])
            n = jnp.tanh(gi[:, 2 * H:] + rz[:, :H] * gh[:, 2 * H:])
            if h is None:
                h = n - rz[:, H:] * n
            else:
                h = n + rz[:, H:] * (h - n)
            h_ref[pl.ds(t * Bs, Bs), :] = h

    gru_steps(w0_ref, ghb_ref[:, 0:H3])

    gi_ref[...] = (jnp.dot(h_ref[...], wih1_ref[...],
                           preferred_element_type=jnp.float32)
                   + bias1i_ref[...])

    gru_steps(w1_ref, ghb_ref[:, H3:2 * H3])

    logits = (jnp.dot(h_ref[...], wlin_ref[...],
                      preferred_element_type=jnp.float32) + blin_ref[...])
    rows = [logits[t * Bs:(t + 1) * Bs, :] for t in range(T)]
    m = functools.reduce(jnp.maximum, rows)
    tot = functools.reduce(lambda a, b: a + b,
                           [jnp.exp(r - m) for r in rows])
    lse = m + jnp.log(tot)
    for t in range(T):
        o_ref[:, pl.ds(t, 1), :] = (rows[t] - lse)[:, None, :]


def kernel(tokens, table, wih0, bias0i, whh_blk, gh_bias, wih1, bias1i,
           w_lin, b_lin):
    B, T = tokens.shape
    V, E = table.shape
    C = w_lin.shape[1]
    H3 = wih0.shape[1]
    H = H3 // 3
    S = E // 128
    G = 2 if (B % 2 == 0 and (B // 2) % 8 == 0) else 1
    Bs = B // G

    table3 = table.reshape(V, S, 128)

    def cs(arr):
        nd = arr.ndim
        return pl.BlockSpec(arr.shape, lambda i, tok, _nd=nd: (0,) * _nd)

    grid_spec = pltpu.PrefetchScalarGridSpec(
        num_scalar_prefetch=1,
        grid=(G,),
        in_specs=[
            pl.BlockSpec(memory_space=pl.ANY),
            cs(wih0), cs(bias0i),
            pl.BlockSpec((H, H3), lambda i, tok: (0, 0)),
            pl.BlockSpec((H, H3), lambda i, tok: (1, 1)),
            cs(gh_bias), cs(wih1), cs(bias1i), cs(w_lin), cs(b_lin),
        ],
        out_specs=pl.BlockSpec((Bs, T, C), lambda i, tok: (i, 0, 0)),
        scratch_shapes=[
            pltpu.VMEM((Bs * T, S, 128), jnp.float32),
            pltpu.VMEM((Bs * T, H3), jnp.float32),
            pltpu.VMEM((Bs * T, H), jnp.float32),
            pltpu.SemaphoreType.DMA,
        ],
    )
    return pl.pallas_call(
        _slot_rnn_fwd,
        out_shape=jax.ShapeDtypeStruct((B, T, C), jnp.float32),
        grid_spec=grid_spec,
        compiler_params=pltpu.CompilerParams(
            dimension_semantics=("parallel",)),
    )(tokens, table3, wih0, bias0i, whh_blk, whh_blk, gh_bias,
      wih1, bias1i, w_lin, b_lin)


# speedup vs baseline: 1.9053x; 1.9053x over previous
import functools

import jax
import jax.numpy as jnp
from jax import lax
from jax.experimental import pallas as pl
from jax.experimental.pallas import tpu as pltpu


def _slot_rnn_fwd(tok_sm, table_hbm, wih0_ref, bias0i_ref, w0_ref, w1_ref,
                  ghb_ref, wih1_ref, bias1i_ref, wlin_ref, blin_ref,
                  o_ref, slab_ref, emb_ref, gi_ref, h_ref, sem):
    i = pl.program_id(0)
    Bs, T, C = o_ref.shape
    E = emb_ref.shape[1]
    H3 = wih0_ref.shape[1]
    H = H3 // 3
    M = Bs * T

    toks = []
    for t in range(T):
        for b in range(Bs):
            tok = tok_sm[i * Bs + b, t]
            toks.append(tok)
            base = pl.multiple_of((tok >> 3) << 3, 8)
            pltpu.make_async_copy(
                table_hbm.at[pl.ds(base, 8), :],
                slab_ref.at[t * Bs + b], sem).start()
    pltpu.make_async_copy(
        slab_ref.at[pl.ds(0, M)], slab_ref.at[pl.ds(0, M)], sem).wait()

    iota8 = lax.broadcasted_iota(jnp.int32, (8, E), 0)
    for slot in range(M):
        sel = (iota8 == (toks[slot] & 7)).astype(jnp.float32)
        emb_ref[pl.ds(slot, 1), :] = jnp.sum(
            slab_ref[slot] * sel, axis=0, keepdims=True)

    gi_ref[...] = (jnp.dot(emb_ref[...], wih0_ref[...],
                           preferred_element_type=jnp.float32)
                   + bias0i_ref[...])

    def gru_steps(w_ref, gb):
        w = w_ref[...]
        h = None
        for t in range(T):
            gi = gi_ref[pl.ds(t * Bs, Bs), :]
            if h is None:
                gh = jnp.broadcast_to(gb, (Bs, H3))
            else:
                gh = jnp.dot(h, w, preferred_element_type=jnp.float32) + gb
            rz = jax.nn.sigmoid(gi[:, :2 * H] + gh[:, :2 * H])
            n = jnp.tanh(gi[:, 2 * H:] + rz[:, :H] * gh[:, 2 * H:])
            if h is None:
                h = n - rz[:, H:] * n
            else:
                h = n + rz[:, H:] * (h - n)
            h_ref[pl.ds(t * Bs, Bs), :] = h

    gru_steps(w0_ref, ghb_ref[:, 0:H3])

    gi_ref[...] = (jnp.dot(h_ref[...], wih1_ref[...],
                           preferred_element_type=jnp.float32)
                   + bias1i_ref[...])

    gru_steps(w1_ref, ghb_ref[:, H3:2 * H3])

    logits = (jnp.dot(h_ref[...], wlin_ref[...],
                      preferred_element_type=jnp.float32) + blin_ref[...])
    rows = [logits[t * Bs:(t + 1) * Bs, :] for t in range(T)]
    m = functools.reduce(jnp.maximum, rows)
    tot = functools.reduce(lambda a, b: a + b,
                           [jnp.exp(r - m) for r in rows])
    lse = m + jnp.log(tot)
    for t in range(T):
        o_ref[:, pl.ds(t, 1), :] = (rows[t] - lse)[:, None, :]


def kernel(tokens, table, wih0, bias0i, whh_blk, gh_bias, wih1, bias1i,
           w_lin, b_lin):
    B, T = tokens.shape
    V, E = table.shape
    C = w_lin.shape[1]
    H3 = wih0.shape[1]
    H = H3 // 3
    G = 2 if (B % 2 == 0 and (B // 2) % 8 == 0) else 1
    Bs = B // G

    def cs(arr):
        nd = arr.ndim
        return pl.BlockSpec(arr.shape, lambda i, tok, _nd=nd: (0,) * _nd)

    grid_spec = pltpu.PrefetchScalarGridSpec(
        num_scalar_prefetch=1,
        grid=(G,),
        in_specs=[
            pl.BlockSpec(memory_space=pl.ANY),
            cs(wih0), cs(bias0i),
            pl.BlockSpec((H, H3), lambda i, tok: (0, 0)),
            pl.BlockSpec((H, H3), lambda i, tok: (1, 1)),
            cs(gh_bias), cs(wih1), cs(bias1i), cs(w_lin), cs(b_lin),
        ],
        out_specs=pl.BlockSpec((Bs, T, C), lambda i, tok: (i, 0, 0)),
        scratch_shapes=[
            pltpu.VMEM((Bs * T, 8, E), jnp.float32),
            pltpu.VMEM((Bs * T, E), jnp.float32),
            pltpu.VMEM((Bs * T, H3), jnp.float32),
            pltpu.VMEM((Bs * T, H), jnp.float32),
            pltpu.SemaphoreType.DMA,
        ],
    )
    return pl.pallas_call(
        _slot_rnn_fwd,
        out_shape=jax.ShapeDtypeStruct((B, T, C), jnp.float32),
        grid_spec=grid_spec,
        compiler_params=pltpu.CompilerParams(
            dimension_semantics=("parallel",)),
    )(tokens, table, wih0, bias0i, whh_blk, whh_blk, gh_bias,
      wih1, bias1i, w_lin, b_lin)


# speedup vs baseline: 2.8939x; 1.5189x over previous
import functools

import jax
import jax.numpy as jnp
from jax import lax
from jax.experimental import pallas as pl
from jax.experimental.pallas import tpu as pltpu


def _slot_rnn_fwd(tok_sm, table_hbm, wih0_ref, bias0i_ref, w0_ref, w1_ref,
                  ghb_ref, wih1_ref, bias1i_ref, wlin_ref, blin_ref,
                  o_ref, slab_ref, emb_ref, gi_ref, h_ref, sem):
    i = pl.program_id(0)
    Bs, T, C = o_ref.shape
    E = emb_ref.shape[1]
    H3 = wih0_ref.shape[1]
    H = H3 // 3
    M = Bs * T

    toks = []
    for t in range(T):
        for b in range(Bs):
            tok = tok_sm[i * Bs + b, t]
            toks.append(tok)
            base = pl.multiple_of((tok >> 3) << 3, 8)
            pltpu.make_async_copy(
                table_hbm.at[pl.ds(base, 8), :],
                slab_ref.at[t * Bs + b], sem).start()
    pltpu.make_async_copy(
        slab_ref.at[pl.ds(0, M)], slab_ref.at[pl.ds(0, M)], sem).wait()

    iota8 = lax.broadcasted_iota(jnp.int32, (8, E), 0)
    for slot in range(M):
        sel = (iota8 == (toks[slot] & 7)).astype(jnp.float32)
        emb_ref[pl.ds(slot, 1), :] = jnp.sum(
            slab_ref[slot] * sel, axis=0, keepdims=True)

    gi_ref[...] = (jnp.dot(emb_ref[...], wih0_ref[...],
                           preferred_element_type=jnp.float32)
                   + bias0i_ref[...])

    def gru_steps(w_ref, gb):
        w = w_ref[...]
        h = None
        for t in range(T):
            gi = gi_ref[pl.ds(t * Bs, Bs), :]
            if h is None:
                gh = jnp.broadcast_to(gb, (Bs, H3))
            else:
                gh = jnp.dot(h, w, preferred_element_type=jnp.float32) + gb
            rz = jax.nn.sigmoid(gi[:, :2 * H] + gh[:, :2 * H])
            n = jnp.tanh(gi[:, 2 * H:] + rz[:, :H] * gh[:, 2 * H:])
            if h is None:
                h = n - rz[:, H:] * n
            else:
                h = n + rz[:, H:] * (h - n)
            h_ref[pl.ds(t * Bs, Bs), :] = h

    gru_steps(w0_ref, ghb_ref[:, 0:H3])

    gi_ref[...] = (jnp.dot(h_ref[...], wih1_ref[...],
                           preferred_element_type=jnp.float32)
                   + bias1i_ref[...])

    gru_steps(w1_ref, ghb_ref[:, H3:2 * H3])

    logits = (jnp.dot(h_ref[...], wlin_ref[...],
                      preferred_element_type=jnp.float32) + blin_ref[...])
    rows = [logits[t * Bs:(t + 1) * Bs, :] for t in range(T)]
    m = functools.reduce(jnp.maximum, rows)
    tot = functools.reduce(lambda a, b: a + b,
                           [jnp.exp(r - m) for r in rows])
    lse = m + jnp.log(tot)
    for t in range(T):
        o_ref[:, pl.ds(t, 1), :] = (rows[t] - lse)[:, None, :]


def kernel(tokens, table, wih0, bias0i, whh_blk, gh_bias, wih1, bias1i,
           w_lin, b_lin):
    B, T = tokens.shape
    V, E = table.shape
    C = w_lin.shape[1]
    H3 = wih0.shape[1]
    H = H3 // 3
    G = 1
    Bs = B // G

    def cs(arr):
        nd = arr.ndim
        return pl.BlockSpec(arr.shape, lambda i, tok, _nd=nd: (0,) * _nd)

    grid_spec = pltpu.PrefetchScalarGridSpec(
        num_scalar_prefetch=1,
        grid=(G,),
        in_specs=[
            pl.BlockSpec(memory_space=pl.ANY),
            cs(wih0), cs(bias0i),
            pl.BlockSpec((H, H3), lambda i, tok: (0, 0)),
            pl.BlockSpec((H, H3), lambda i, tok: (1, 1)),
            cs(gh_bias), cs(wih1), cs(bias1i), cs(w_lin), cs(b_lin),
        ],
        out_specs=pl.BlockSpec((Bs, T, C), lambda i, tok: (i, 0, 0)),
        scratch_shapes=[
            pltpu.VMEM((Bs * T, 8, E), jnp.float32),
            pltpu.VMEM((Bs * T, E), jnp.float32),
            pltpu.VMEM((Bs * T, H3), jnp.float32),
            pltpu.VMEM((Bs * T, H), jnp.float32),
            pltpu.SemaphoreType.DMA,
        ],
    )
    return pl.pallas_call(
        _slot_rnn_fwd,
        out_shape=jax.ShapeDtypeStruct((B, T, C), jnp.float32),
        grid_spec=grid_spec,
        compiler_params=pltpu.CompilerParams(
            dimension_semantics=("arbitrary",)),
    )(tokens, table, wih0, bias0i, whh_blk, whh_blk, gh_bias,
      wih1, bias1i, w_lin, b_lin)


# speedup vs baseline: 3.0133x; 1.0413x over previous
import functools

import jax
import jax.numpy as jnp
from jax import lax
from jax.experimental import pallas as pl
from jax.experimental.pallas import tpu as pltpu


def _slot_rnn_fwd(tok_sm, table_hbm, whh_hbm, wih0_hbm, wih1_hbm, wlin_hbm,
                  bias0i_ref, ghb_ref, bias1i_ref, blin_ref,
                  o_ref, slab_ref, emb_ref, gi_ref, h_ref,
                  wih0_ref, w0_ref, w1_ref, wih1_ref, wlin_ref,
                  gsem, wsem):
    i = pl.program_id(0)
    Bs, T, C = o_ref.shape
    E = emb_ref.shape[1]
    H3 = wih0_ref.shape[1]
    H = H3 // 3
    M = Bs * T
    half = M // 2

    toks = []
    for t in range(T):
        for b in range(Bs):
            tok = tok_sm[i * Bs + b, t]
            toks.append(tok)
            base = pl.multiple_of((tok >> 3) << 3, 8)
            slot = t * Bs + b
            pltpu.make_async_copy(
                table_hbm.at[pl.ds(base, 8), :],
                slab_ref.at[slot], gsem.at[slot // half]).start()

    pltpu.make_async_copy(wih0_hbm, wih0_ref, wsem.at[0]).start()
    pltpu.make_async_copy(
        whh_hbm.at[pl.ds(0, H), pl.ds(0, H3)], w0_ref, wsem.at[1]).start()
    pltpu.make_async_copy(
        whh_hbm.at[pl.ds(H, H), pl.ds(H3, H3)], w1_ref, wsem.at[2]).start()
    pltpu.make_async_copy(wih1_hbm, wih1_ref, wsem.at[3]).start()
    pltpu.make_async_copy(wlin_hbm, wlin_ref, wsem.at[4]).start()

    iota8 = lax.broadcasted_iota(jnp.int32, (8, E), 0)
    for piece in range(2):
        pltpu.make_async_copy(
            slab_ref.at[pl.ds(piece * half, half)],
            slab_ref.at[pl.ds(piece * half, half)], gsem.at[piece]).wait()
        for slot in range(piece * half, (piece + 1) * half):
            sel = (iota8 == (toks[slot] & 7)).astype(jnp.float32)
            emb_ref[pl.ds(slot, 1), :] = jnp.sum(
                slab_ref[slot] * sel, axis=0, keepdims=True)

    pltpu.make_async_copy(wih0_ref, wih0_ref, wsem.at[0]).wait()
    gi_ref[...] = (jnp.dot(emb_ref[...], wih0_ref[...],
                           preferred_element_type=jnp.float32)
                   + bias0i_ref[...])

    def gru_steps(w_ref, gb):
        w = w_ref[...]
        h = None
        for t in range(T):
            gi = gi_ref[pl.ds(t * Bs, Bs), :]
            if h is None:
                gh = jnp.broadcast_to(gb, (Bs, H3))
            else:
                gh = jnp.dot(h, w, preferred_element_type=jnp.float32) + gb
            rz = jax.nn.sigmoid(gi[:, :2 * H] + gh[:, :2 * H])
            n = jnp.tanh(gi[:, 2 * H:] + rz[:, :H] * gh[:, 2 * H:])
            if h is None:
                h = n - rz[:, H:] * n
            else:
                h = n + rz[:, H:] * (h - n)
            h_ref[pl.ds(t * Bs, Bs), :] = h

    pltpu.make_async_copy(w0_ref, w0_ref, wsem.at[1]).wait()
    gru_steps(w0_ref, ghb_ref[:, 0:H3])

    pltpu.make_async_copy(wih1_ref, wih1_ref, wsem.at[3]).wait()
    gi_ref[...] = (jnp.dot(h_ref[...], wih1_ref[...],
                           preferred_element_type=jnp.float32)
                   + bias1i_ref[...])

    pltpu.make_async_copy(w1_ref, w1_ref, wsem.at[2]).wait()
    gru_steps(w1_ref, ghb_ref[:, H3:2 * H3])

    pltpu.make_async_copy(wlin_ref, wlin_ref, wsem.at[4]).wait()
    logits = (jnp.dot(h_ref[...], wlin_ref[...],
                      preferred_element_type=jnp.float32) + blin_ref[...])
    rows = [logits[t * Bs:(t + 1) * Bs, :] for t in range(T)]
    m = functools.reduce(jnp.maximum, rows)
    tot = functools.reduce(lambda a, b: a + b,
                           [jnp.exp(r - m) for r in rows])
    lse = m + jnp.log(tot)
    for t in range(T):
        o_ref[:, pl.ds(t, 1), :] = (rows[t] - lse)[:, None, :]


def kernel(tokens, table, wih0, bias0i, whh_blk, gh_bias, wih1, bias1i,
           w_lin, b_lin):
    B, T = tokens.shape
    V, E = table.shape
    C = w_lin.shape[1]
    H3 = wih0.shape[1]
    H = H3 // 3
    G = 1
    Bs = B // G

    def cs(arr):
        nd = arr.ndim
        return pl.BlockSpec(arr.shape, lambda i, tok, _nd=nd: (0,) * _nd)

    hbm = pl.BlockSpec(memory_space=pl.ANY)

    grid_spec = pltpu.PrefetchScalarGridSpec(
        num_scalar_prefetch=1,
        grid=(G,),
        in_specs=[
            hbm,
            hbm,
            hbm,
            hbm,
            hbm,
            cs(bias0i), cs(gh_bias), cs(bias1i), cs(b_lin),
        ],
        out_specs=pl.BlockSpec((Bs, T, C), lambda i, tok: (i, 0, 0)),
        scratch_shapes=[
            pltpu.VMEM((Bs * T, 8, E), jnp.float32),
            pltpu.VMEM((Bs * T, E), jnp.float32),
            pltpu.VMEM((Bs * T, H3), jnp.float32),
            pltpu.VMEM((Bs * T, H), jnp.float32),
            pltpu.VMEM((E, H3), jnp.float32),
            pltpu.VMEM((H, H3), jnp.float32),
            pltpu.VMEM((H, H3), jnp.float32),
            pltpu.VMEM((H, H3), jnp.float32),
            pltpu.VMEM((H, C), jnp.float32),
            pltpu.SemaphoreType.DMA((2,)),
            pltpu.SemaphoreType.DMA((5,)),
        ],
    )
    return pl.pallas_call(
        _slot_rnn_fwd,
        out_shape=jax.ShapeDtypeStruct((B, T, C), jnp.float32),
        grid_spec=grid_spec,
        compiler_params=pltpu.CompilerParams(
            dimension_semantics=("arbitrary",)),
    )(tokens, table, whh_blk, wih0, wih1, w_lin,
      bias0i, gh_bias, bias1i, b_lin)


# speedup vs baseline: 3.5420x; 1.1754x over previous
import functools

import jax
import jax.numpy as jnp
from jax import lax
from jax.experimental import pallas as pl
from jax.experimental.pallas import tpu as pltpu


def _slot_rnn_fwd(tok_sm, table_hbm, whh_hbm, wih0_hbm, wih1_hbm, wlin_hbm,
                  bias0i_ref, ghb_ref, bias1i_ref, blin_ref,
                  o_ref, slab_ref, emb_ref, gi_ref, h1_ref,
                  wih0_ref, w0_ref, w1_ref, wih1_ref, wlin_ref,
                  gsem, wsem):
    i = pl.program_id(0)
    Bs, T, C = o_ref.shape
    E = emb_ref.shape[1]
    H3 = wih0_ref.shape[1]
    H = H3 // 3
    M = Bs * T
    half = M // 2
    f32 = jnp.float32

    toks = []
    for t in range(T):
        for b in range(Bs):
            tok = tok_sm[i * Bs + b, t]
            toks.append(tok)
            base = pl.multiple_of((tok >> 3) << 3, 8)
            slot = t * Bs + b
            pltpu.make_async_copy(
                table_hbm.at[pl.ds(base, 8), :],
                slab_ref.at[slot], gsem.at[slot // half]).start()

    pltpu.make_async_copy(wih0_hbm, wih0_ref, wsem.at[0]).start()
    pltpu.make_async_copy(
        whh_hbm.at[pl.ds(0, H), pl.ds(0, H3)], w0_ref, wsem.at[1]).start()
    pltpu.make_async_copy(
        whh_hbm.at[pl.ds(H, H), pl.ds(H3, H3)], w1_ref, wsem.at[2]).start()
    pltpu.make_async_copy(wih1_hbm, wih1_ref, wsem.at[3]).start()
    pltpu.make_async_copy(wlin_hbm, wlin_ref, wsem.at[4]).start()

    iota8 = lax.broadcasted_iota(jnp.int32, (8, E), 0)
    for piece in range(2):
        pltpu.make_async_copy(
            slab_ref.at[pl.ds(piece * half, half)],
            slab_ref.at[pl.ds(piece * half, half)], gsem.at[piece]).wait()
        for slot in range(piece * half, (piece + 1) * half):
            sel = (iota8 == (toks[slot] & 7)).astype(f32)
            emb_ref[pl.ds(slot, 1), :] = jnp.sum(
                slab_ref[slot] * sel, axis=0, keepdims=True)

    pltpu.make_async_copy(wih0_ref, wih0_ref, wsem.at[0]).wait()
    gi_ref[...] = (jnp.dot(emb_ref[...], wih0_ref[...],
                           preferred_element_type=f32)
                   + bias0i_ref[...])

    pltpu.make_async_copy(w0_ref, w0_ref, wsem.at[1]).wait()
    pltpu.make_async_copy(wih1_ref, wih1_ref, wsem.at[3]).wait()
    pltpu.make_async_copy(w1_ref, w1_ref, wsem.at[2]).wait()
    w0 = w0_ref[...]
    w1 = w1_ref[...]
    wih1 = wih1_ref[...]
    gb0 = ghb_ref[:, 0:H3]
    gb1 = ghb_ref[:, H3:2 * H3]
    b1i = bias1i_ref[...]

    def cell(gi, gh, h):
        rz = jax.nn.sigmoid(gi[:, :2 * H] + gh[:, :2 * H])
        n = jnp.tanh(gi[:, 2 * H:] + rz[:, :H] * gh[:, 2 * H:])
        if h is None:
            return n - rz[:, H:] * n
        return n + rz[:, H:] * (h - n)

    gb0_b = jnp.broadcast_to(gb0, (Bs, H3))
    gb1_b = jnp.broadcast_to(gb1, (Bs, H3))
    h0 = cell(gi_ref[pl.ds(0, Bs), :], gb0_b, None)
    gi1 = [jnp.dot(h0, wih1, preferred_element_type=f32) + b1i]
    h1 = None
    for t in range(1, T):
        gh0 = jnp.dot(h0, w0, preferred_element_type=f32) + gb0
        if t == 1:
            gh1 = gb1_b
        else:
            gh1 = jnp.dot(h1, w1, preferred_element_type=f32) + gb1
        h1 = cell(gi1[t - 1], gh1, h1)
        h1_ref[pl.ds((t - 1) * Bs, Bs), :] = h1
        h0 = cell(gi_ref[pl.ds(t * Bs, Bs), :], gh0, h0)
        gi1.append(jnp.dot(h0, wih1, preferred_element_type=f32) + b1i)
    gh1 = jnp.dot(h1, w1, preferred_element_type=f32) + gb1
    h1 = cell(gi1[T - 1], gh1, h1)
    h1_ref[pl.ds((T - 1) * Bs, Bs), :] = h1

    pltpu.make_async_copy(wlin_ref, wlin_ref, wsem.at[4]).wait()
    logits = (jnp.dot(h1_ref[...], wlin_ref[...],
                      preferred_element_type=f32) + blin_ref[...])
    rows = [logits[t * Bs:(t + 1) * Bs, :] for t in range(T)]
    m = functools.reduce(jnp.maximum, rows)
    tot = functools.reduce(lambda a, b: a + b,
                           [jnp.exp(r - m) for r in rows])
    lse = m + jnp.log(tot)
    for t in range(T):
        o_ref[:, pl.ds(t, 1), :] = (rows[t] - lse)[:, None, :]


def kernel(tokens, table, wih0, bias0i, whh_blk, gh_bias, wih1, bias1i,
           w_lin, b_lin):
    B, T = tokens.shape
    V, E = table.shape
    C = w_lin.shape[1]
    H3 = wih0.shape[1]
    H = H3 // 3
    G = 1
    Bs = B // G

    def cs(arr):
        nd = arr.ndim
        return pl.BlockSpec(arr.shape, lambda i, tok, _nd=nd: (0,) * _nd)

    hbm = pl.BlockSpec(memory_space=pl.ANY)

    grid_spec = pltpu.PrefetchScalarGridSpec(
        num_scalar_prefetch=1,
        grid=(G,),
        in_specs=[
            hbm,
            hbm,
            hbm,
            hbm,
            hbm,
            cs(bias0i), cs(gh_bias), cs(bias1i), cs(b_lin),
        ],
        out_specs=pl.BlockSpec((Bs, T, C), lambda i, tok: (i, 0, 0)),
        scratch_shapes=[
            pltpu.VMEM((Bs * T, 8, E), jnp.float32),
            pltpu.VMEM((Bs * T, E), jnp.float32),
            pltpu.VMEM((Bs * T, H3), jnp.float32),
            pltpu.VMEM((Bs * T, H), jnp.float32),
            pltpu.VMEM((E, H3), jnp.float32),
            pltpu.VMEM((H, H3), jnp.float32),
            pltpu.VMEM((H, H3), jnp.float32),
            pltpu.VMEM((H, H3), jnp.float32),
            pltpu.VMEM((H, C), jnp.float32),
            pltpu.SemaphoreType.DMA((2,)),
            pltpu.SemaphoreType.DMA((5,)),
        ],
    )
    return pl.pallas_call(
        _slot_rnn_fwd,
        out_shape=jax.ShapeDtypeStruct((B, T, C), jnp.float32),
        grid_spec=grid_spec,
        compiler_params=pltpu.CompilerParams(
            dimension_semantics=("arbitrary",)),
    )(tokens, table, whh_blk, wih0, wih1, w_lin,
      bias0i, gh_bias, bias1i, b_lin)
